# baseline (device time: 13926 ns/iter reference)
import functools

import jax
import jax.numpy as jnp
from jax import lax
from jax.experimental import pallas as pl
from jax.experimental.pallas import tpu as pltpu

N_DEV = 8
N_TOK = 256
D_IN = 128
D_OUT = 256
N_EXP = 16
EXP_PER_DEV = N_EXP // N_DEV
CAP = 12
ROWS = N_TOK // N_DEV


def kernel(x, router_W, route_idx, expert_W):
    def body(x_ref, rw_ref, idx_ref, ew_ref, out_ref,
             acc_ref, recv_ref, send_sems, recv_sems):
        d = lax.axis_index("i")

        barrier_sem = pltpu.get_barrier_semaphore()
        for o in range(1, N_DEV):
            pl.semaphore_signal(
                barrier_sem, inc=1,
                device_id=((d + o) % N_DEV,),
                device_id_type=pl.DeviceIdType.MESH,
            )
        pl.semaphore_wait(barrier_sem, N_DEV - 1)

        e = idx_ref[:, :]
        exp_iota = lax.broadcasted_iota(jnp.int32, (N_TOK, N_EXP), 1)
        oh = (e == exp_iota).astype(jnp.float32)
        row_i = lax.broadcasted_iota(jnp.int32, (N_TOK, N_TOK), 0)
        col_i = lax.broadcasted_iota(jnp.int32, (N_TOK, N_TOK), 1)
        tril = (row_i >= col_i).astype(jnp.float32)
        pos = jnp.dot(tril, oh, preferred_element_type=jnp.float32)
        my_pos = jnp.sum(pos * oh, axis=1, keepdims=True)
        keep = my_pos <= float(CAP)

        xv = x_ref[:, :]
        contrib = jnp.zeros((N_TOK, D_OUT), jnp.float32)
        for k in range(EXP_PER_DEV):
            eid = d * EXP_PER_DEV + k
            m = jnp.where(jnp.logical_and(e == eid, keep), 1.0, 0.0)
            xm = (xv * m).astype(jnp.bfloat16)
            wk = ew_ref[k, :, :].astype(jnp.bfloat16)
            contrib = contrib + jnp.dot(
                xm, wk, preferred_element_type=jnp.float32
            )
        acc_ref[:, :] = contrib

        rdmas = []
        for o in range(1, N_DEV):
            t = (d + o) % N_DEV
            rdma = pltpu.make_async_remote_copy(
                src_ref=acc_ref.at[pl.ds(t * ROWS, ROWS), :],
                dst_ref=recv_ref.at[d],
                send_sem=send_sems.at[t],
                recv_sem=recv_sems.at[d],
                device_id=(t,),
                device_id_type=pl.DeviceIdType.MESH,
            )
            rdma.start()
            rdmas.append(rdma)

        result = acc_ref[pl.ds(d * ROWS, ROWS), :]
        for o in range(1, N_DEV):
            s = (d + o) % N_DEV
            recv = pltpu.make_async_remote_copy(
                src_ref=acc_ref.at[pl.ds(0, ROWS), :],
                dst_ref=recv_ref.at[s],
                send_sem=send_sems.at[s],
                recv_sem=recv_sems.at[s],
                device_id=(s,),
                device_id_type=pl.DeviceIdType.MESH,
            )
            recv.wait_recv()
            result = result + recv_ref[s, :, :]
        out_ref[:, :] = result

        for rdma in rdmas:
            rdma.wait_send()

        @functools.partial(pl.run_scoped, sem2=pltpu.SemaphoreType.REGULAR)
        def _(sem2):
            for o in range(1, N_DEV):
                pl.semaphore_signal(
                    sem2, inc=1,
                    device_id=((d + o) % N_DEV,),
                    device_id_type=pl.DeviceIdType.MESH,
                )
            pl.semaphore_wait(sem2, N_DEV - 1)

    return pl.pallas_call(
        body,
        out_shape=jax.ShapeDtypeStruct((ROWS, D_OUT), jnp.float32),
        in_specs=[pl.BlockSpec(memory_space=pltpu.VMEM)] * 4,
        out_specs=pl.BlockSpec(memory_space=pltpu.VMEM),
        scratch_shapes=[
            pltpu.VMEM((N_TOK, D_OUT), jnp.float32),
            pltpu.VMEM((N_DEV, ROWS, D_OUT), jnp.float32),
            pltpu.SemaphoreType.DMA((N_DEV,)),
            pltpu.SemaphoreType.DMA((N_DEV,)),
        ],
        compiler_params=pltpu.CompilerParams(collective_id=0),
    )(x, router_W, route_idx, expert_W)


# device time: 13156 ns/iter; 1.0585x vs baseline; 1.0585x over previous
import functools

import jax
import jax.numpy as jnp
from jax import lax
from jax.experimental import pallas as pl
from jax.experimental.pallas import tpu as pltpu

N_DEV = 8
N_TOK = 256
D_IN = 128
D_OUT = 256
N_EXP = 16
EXP_PER_DEV = N_EXP // N_DEV
CAP = 12
ROWS = N_TOK // N_DEV


def kernel(x, router_W, route_idx, expert_W):
    def body(x_ref, rw_ref, idx_ref, ew_ref, out_ref,
             acc_ref, recv_ref, send_sems, recv_sems):
        d = lax.axis_index("i")

        e = idx_ref[:, :]
        exp_iota = lax.broadcasted_iota(jnp.int32, (N_TOK, N_EXP), 1)
        oh = (e == exp_iota).astype(jnp.float32)
        row_i = lax.broadcasted_iota(jnp.int32, (N_TOK, N_TOK), 0)
        col_i = lax.broadcasted_iota(jnp.int32, (N_TOK, N_TOK), 1)
        tril = (row_i >= col_i).astype(jnp.float32)
        pos = jnp.dot(tril, oh, preferred_element_type=jnp.float32)
        my_pos = jnp.sum(pos * oh, axis=1, keepdims=True)
        keep = my_pos <= float(CAP)

        xv = x_ref[:, :]
        masked = []
        for k in range(EXP_PER_DEV):
            eid = d * EXP_PER_DEV + k
            m = jnp.where(jnp.logical_and(e == eid, keep), 1.0, 0.0)
            masked.append((xv * m).astype(jnp.bfloat16))
        xm = jnp.concatenate(masked, axis=1)
        wk = jnp.concatenate(
            [ew_ref[k, :, :] for k in range(EXP_PER_DEV)], axis=0
        ).astype(jnp.bfloat16)
        contrib = jnp.dot(xm, wk, preferred_element_type=jnp.float32)
        acc_ref[:, :] = contrib.astype(jnp.bfloat16)

        barrier_sem = pltpu.get_barrier_semaphore()
        for o in range(1, N_DEV):
            pl.semaphore_signal(
                barrier_sem, inc=1,
                device_id=((d + o) % N_DEV,),
                device_id_type=pl.DeviceIdType.MESH,
            )
        pl.semaphore_wait(barrier_sem, N_DEV - 1)

        rdmas = []
        for o in range(1, N_DEV):
            t = (d + o) % N_DEV
            rdma = pltpu.make_async_remote_copy(
                src_ref=acc_ref.at[pl.ds(t * ROWS, ROWS), :],
                dst_ref=recv_ref.at[d],
                send_sem=send_sems.at[t],
                recv_sem=recv_sems.at[d],
                device_id=(t,),
                device_id_type=pl.DeviceIdType.MESH,
            )
            rdma.start()
            rdmas.append(rdma)

        result = acc_ref[pl.ds(d * ROWS, ROWS), :].astype(jnp.float32)
        for o in range(1, N_DEV):
            s = (d + o) % N_DEV
            recv = pltpu.make_async_remote_copy(
                src_ref=acc_ref.at[pl.ds(0, ROWS), :],
                dst_ref=recv_ref.at[s],
                send_sem=send_sems.at[s],
                recv_sem=recv_sems.at[s],
                device_id=(s,),
                device_id_type=pl.DeviceIdType.MESH,
            )
            recv.wait_recv()
            result = result + recv_ref[s, :, :].astype(jnp.float32)
        out_ref[:, :] = result

        for rdma in rdmas:
            rdma.wait_send()

        @functools.partial(pl.run_scoped, sem2=pltpu.SemaphoreType.REGULAR)
        def _(sem2):
            for o in range(1, N_DEV):
                pl.semaphore_signal(
                    sem2, inc=1,
                    device_id=((d + o) % N_DEV,),
                    device_id_type=pl.DeviceIdType.MESH,
                )
            pl.semaphore_wait(sem2, N_DEV - 1)

    return pl.pallas_call(
        body,
        out_shape=jax.ShapeDtypeStruct((ROWS, D_OUT), jnp.float32),
        in_specs=[pl.BlockSpec(memory_space=pltpu.VMEM)] * 4,
        out_specs=pl.BlockSpec(memory_space=pltpu.VMEM),
        scratch_shapes=[
            pltpu.VMEM((N_TOK, D_OUT), jnp.bfloat16),
            pltpu.VMEM((N_DEV, ROWS, D_OUT), jnp.bfloat16),
            pltpu.SemaphoreType.DMA((N_DEV,)),
            pltpu.SemaphoreType.DMA((N_DEV,)),
        ],
        compiler_params=pltpu.CompilerParams(collective_id=0),
    )(x, router_W, route_idx, expert_W)
